# Initial kernel scaffold; baseline (speedup 1.0000x reference)
import jax
import jax.numpy as jnp
from jax.experimental import pallas as pl


def kernel(xyzs, feats, up0_W1, up0_b1, up0_Wx, up0_bx, up0_Wf, up0_bf, num0_W1, num0_b1, num0_W2, num0_b2, ref0_W1, ref0_b1, ref0_Wf, ref0_bf, ref0_Wx, ref0_bx, up1_W1, up1_b1, up1_Wx, up1_bx, up1_Wf, up1_bf, num1_W1, num1_b1, num1_W2, num1_b2, ref1_W1, ref1_b1, ref1_Wf, ref1_bf, ref1_Wx, ref1_bx):
    raise NotImplementedError("write your pallas kernel here")


# speedup vs baseline: 13.0316x; 13.0316x over previous
import functools

import jax
import jax.numpy as jnp
from jax.experimental import pallas as pl

B = 1
N0 = 512
DIM = 128
HID = 256
RC = 8
RS = 4
K = 16

_DOT = functools.partial(
    jax.lax.dot_general, precision=jax.lax.Precision.DEFAULT,
    preferred_element_type=jnp.float32)


def _mm(a, b):
    return _DOT(a, b, (((1,), (0,)), ((), ())))


def _layer_body(xyz_ref, f_ref, w1_ref, b1_ref, wx_ref, bx_ref, wf_ref,
                bf_ref, nw1_ref, nb1_ref, nw2_ref, nb2_ref, rw1_ref, rb1_ref,
                rwf_ref, rbf_ref, rwx_ref, rbx_ref,
                xyz_out_ref, f_out_ref, un_out_ref):
    x = xyz_ref[...]
    f = f_ref[...]
    h = jnp.maximum(_mm(f, w1_ref[...]) + b1_ref[...], 0.0)
    co = _mm(h, wx_ref[...]) + bx_ref[...]
    cf = _mm(h, wf_ref[...]) + bf_ref[...]
    hn = jnp.maximum(_mm(f, nw1_ref[...]) + nb1_ref[...], 0.0)
    logit = _mm(hn, nw2_ref[...]) + nb2_ref[...]
    un = 1.0 + (RC - 1.0) * jax.nn.sigmoid(logit[:, 0:1])
    un_out_ref[...] = un
    for r in range(RS):
        xyz_c = x + co[:, r * 128:(r + 1) * 128]
        m_r = jax.nn.sigmoid(un - (r + 1.0))
        f_c = cf[:, r * 128:(r + 1) * 128] * m_r
        h2 = jnp.maximum(_mm(f_c, rw1_ref[...]) + rb1_ref[...], 0.0)
        f_out_ref[:, r, :] = f_c + _mm(h2, rwf_ref[...]) + rbf_ref[...]
        xyz_out_ref[:, r, :] = xyz_c + _mm(h2, rwx_ref[...]) + rbx_ref[...]


def _run_layer(xyzp, f, w1, b1, wxb, bxb, wf4, bf4, nw1, nb1, nw2p, nb2p,
               rw1, rb1, rwf, rbf, rwxp, rbxp, bn):
    n = f.shape[0]
    grid = (n // bn,)
    row = lambda i: (i, 0)
    row3 = lambda i: (i, 0, 0)
    full2 = lambda i: (0, 0)
    wspec = lambda a: pl.BlockSpec(a.shape, full2)
    in_specs = [
        pl.BlockSpec((bn, 128), row), pl.BlockSpec((bn, 128), row),
        wspec(w1), wspec(b1), wspec(wxb), wspec(bxb), wspec(wf4), wspec(bf4),
        wspec(nw1), wspec(nb1), wspec(nw2p), wspec(nb2p),
        wspec(rw1), wspec(rb1), wspec(rwf), wspec(rbf), wspec(rwxp),
        wspec(rbxp),
    ]
    out_shape = [
        jax.ShapeDtypeStruct((n, RS, 128), jnp.float32),
        jax.ShapeDtypeStruct((n, RS, 128), jnp.float32),
        jax.ShapeDtypeStruct((n, 1), jnp.float32),
    ]
    out_specs = [
        pl.BlockSpec((bn, RS, 128), row3),
        pl.BlockSpec((bn, RS, 128), row3),
        pl.BlockSpec((bn, 1), row),
    ]
    xyz_out, f_out, un = pl.pallas_call(
        _layer_body, grid=grid, in_specs=in_specs, out_specs=out_specs,
        out_shape=out_shape)(
            xyzp, f, w1, b1, wxb, bxb, wf4, bf4, nw1, nb1, nw2p, nb2p,
            rw1, rb1, rwf, rbf, rwxp, rbxp)
    return (xyz_out.reshape(n * RS, 128), f_out.reshape(n * RS, 128), un)


def _knn_body(prev_ref, currt_ref, un_ref, mdis_ref):
    pv = prev_ref[...]
    ct = currt_ref[...]
    pn = jnp.sum(pv * pv, axis=1, keepdims=True)
    cn = jnp.sum(ct * ct, axis=0, keepdims=True)
    d2 = pn + cn - 2.0 * _mm(pv, ct)
    big = jnp.float32(3.0e38)
    acc = jnp.zeros_like(pn)
    rem = jnp.full_like(pn, float(K))
    vals = d2
    for _ in range(K):
        m = jnp.min(vals, axis=1, keepdims=True)
        eq = vals == m
        c = jnp.sum(eq.astype(jnp.float32), axis=1, keepdims=True)
        t = jnp.minimum(c, rem)
        acc = acc + jnp.where(
            t > 0.0, jnp.sqrt(jnp.maximum(m, 1e-12)) * t, 0.0)
        rem = rem - t
        vals = jnp.where(eq, big, vals)
    mdis_ref[...] = acc / un_ref[...]


def _run_knn(prevp, currp, un, bp):
    p = prevp.shape[0]
    c = currp.shape[0]
    currt = currp.T
    grid = (p // bp,)
    row = lambda i: (i, 0)
    full2 = lambda i: (0, 0)
    mdis = pl.pallas_call(
        _knn_body, grid=grid,
        in_specs=[pl.BlockSpec((bp, 128), row),
                  pl.BlockSpec((128, c), full2),
                  pl.BlockSpec((bp, 1), row)],
        out_specs=pl.BlockSpec((bp, 1), row),
        out_shape=jax.ShapeDtypeStruct((p, 1), jnp.float32),
    )(prevp, currt, un)
    return mdis


def _prep_layer_params(w1, b1, wx, bx, wf, bf, nw1, nb1, nw2, nb2,
                       rw1, rb1, rwf, rbf, rwx, rbx):
    wxb = jnp.zeros((HID, RS, 128), jnp.float32).at[:, :, 0:3].set(
        wx.reshape(HID, RC, 3)[:, :RS, :]).reshape(HID, RS * 128)
    bxb = jnp.zeros((RS, 128), jnp.float32).at[:, 0:3].set(
        bx.reshape(RC, 3)[:RS, :]).reshape(1, RS * 128)
    wf4 = wf[:, :RS * DIM]
    bf4 = bf[:RS * DIM].reshape(1, RS * DIM)
    nw2p = jnp.zeros((HID, 128), jnp.float32).at[:, 0:1].set(nw2)
    nb2p = jnp.zeros((1, 128), jnp.float32).at[0, 0].set(nb2[0])
    rwxp = jnp.zeros((HID, 128), jnp.float32).at[:, 0:3].set(rwx)
    rbxp = jnp.zeros((1, 128), jnp.float32).at[0, 0:3].set(rbx)
    return (w1, b1.reshape(1, HID), wxb, bxb, wf4, bf4,
            nw1, nb1.reshape(1, HID), nw2p, nb2p,
            rw1, rb1.reshape(1, HID), rwf, rbf.reshape(1, DIM), rwxp, rbxp)


def kernel(xyzs, feats, up0_W1, up0_b1, up0_Wx, up0_bx, up0_Wf, up0_bf,
           num0_W1, num0_b1, num0_W2, num0_b2, ref0_W1, ref0_b1, ref0_Wf,
           ref0_bf, ref0_Wx, ref0_bx, up1_W1, up1_b1, up1_Wx, up1_bx,
           up1_Wf, up1_bf, num1_W1, num1_b1, num1_W2, num1_b2, ref1_W1,
           ref1_b1, ref1_Wf, ref1_bf, ref1_Wx, ref1_bx):
    xyz0 = jnp.transpose(xyzs[0])
    xyz0p = jnp.zeros((N0, 128), jnp.float32).at[:, 0:3].set(xyz0)
    f0 = jnp.transpose(feats[0])

    p0 = _prep_layer_params(up0_W1, up0_b1, up0_Wx, up0_bx, up0_Wf, up0_bf,
                            num0_W1, num0_b1, num0_W2, num0_b2, ref0_W1,
                            ref0_b1, ref0_Wf, ref0_bf, ref0_Wx, ref0_bx)
    p1 = _prep_layer_params(up1_W1, up1_b1, up1_Wx, up1_bx, up1_Wf, up1_bf,
                            num1_W1, num1_b1, num1_W2, num1_b2, ref1_W1,
                            ref1_b1, ref1_Wf, ref1_bf, ref1_Wx, ref1_bx)

    xyz1p, f1, un0 = _run_layer(xyz0p, f0, *p0, bn=512)
    xyz2p, f2, un1 = _run_layer(xyz1p, f1, *p1, bn=1024)

    mdis0 = _run_knn(xyz0p, xyz1p, un0, bp=512)
    mdis1 = _run_knn(xyz1p, xyz2p, un1, bp=512)

    xyz1_out = jnp.transpose(xyz1p[:, 0:3])[None]
    xyz2_out = jnp.transpose(xyz2p[:, 0:3])[None]
    f_out = jnp.transpose(f2)[None]
    return (xyz1_out, xyz2_out,
            un0.reshape(1, N0), un1.reshape(1, 4 * N0),
            mdis0.reshape(1, N0), mdis1.reshape(1, 4 * N0),
            f_out)
